# Initial kernel scaffold; baseline (speedup 1.0000x reference)
import functools

import jax
import jax.numpy as jnp
from jax import lax
from jax.experimental import pallas as pl
from jax.experimental.pallas import tpu as pltpu
from jax.experimental.pallas import tpu_sc as plsc

_B = 16384
_F = 26
_V = 100000
_D = 16
_NC = 2
_NS = 16
_NW = _NC * _NS
_L = 16

_ROWS = _B * _F
_RPW = _ROWS // _NW
_CHUNK = 6656
_NCHUNK = _RPW // _CHUNK


def _sc_gather(tab_flat, idx_flat):
    mesh = plsc.VectorSubcoreMesh(
        core_axis_name="c", subcore_axis_name="s",
        num_cores=_NC, num_subcores=_NS)

    @functools.partial(
        pl.kernel,
        out_type=jax.ShapeDtypeStruct((_ROWS, _D), jnp.float32),
        mesh=mesh,
        scratch_types=[
            pltpu.VMEM((_CHUNK,), jnp.int32),
            pltpu.VMEM((_CHUNK, _D), jnp.float32),
            pltpu.SemaphoreType.DMA,
        ],
    )
    def body(tab_hbm, idx_hbm, out_hbm, idx_v, rows_v, sem):
        wid = lax.axis_index("s") * _NC + lax.axis_index("c")
        lane = lax.broadcasted_iota(jnp.int32, (_L,), 0)

        def do_chunk(c, _):
            gbase = wid * _RPW + c * _CHUNK
            pltpu.sync_copy(idx_hbm.at[pl.ds(gbase, _CHUNK)], idx_v)

            def add_off(i, _):
                p = i * _L + lane
                off = lax.rem(p, _F) * _V
                idx_v[pl.ds(i * _L, _L)] = idx_v[pl.ds(i * _L, _L)] + off
                return 0
            lax.fori_loop(0, _CHUNK // _L, add_off, 0, unroll=4)

            pltpu.async_copy(tab_hbm.at[idx_v], rows_v, sem).wait()
            pltpu.sync_copy(rows_v, out_hbm.at[pl.ds(gbase, _CHUNK)])
            return 0

        lax.fori_loop(0, _NCHUNK, do_chunk, 0)

    return body(tab_flat, idx_flat)


def _mlp_body(emb, xc, w1a, w1b, b1, w2, b2, w3, b3, w4, b4, out):
    h = jnp.dot(emb[...], w1a[...], preferred_element_type=jnp.float32)
    h += jnp.dot(xc[...], w1b[...], preferred_element_type=jnp.float32)
    h = jnp.maximum(h + b1[...], 0.0)
    h = jnp.maximum(
        jnp.dot(h, w2[...], preferred_element_type=jnp.float32) + b2[...], 0.0)
    h = jnp.maximum(
        jnp.dot(h, w3[...], preferred_element_type=jnp.float32) + b3[...], 0.0)
    out[...] = jnp.dot(h, w4[...], preferred_element_type=jnp.float32) + b4[...]


def _tc_mlp(emb, xc_pad, w1a, w1b, b1, w2, b2, w3, b3, w4, b4):
    bm = 1024
    in_dim = emb.shape[1]
    h1, h2, h3, od = w1a.shape[1], w2.shape[1], w3.shape[1], w4.shape[1]
    xk = xc_pad.shape[1]
    grid = (_B // bm,)
    full = lambda r, c: pl.BlockSpec((r, c), lambda i: (0, 0))
    return pl.pallas_call(
        _mlp_body,
        grid=grid,
        in_specs=[
            pl.BlockSpec((bm, in_dim), lambda i: (i, 0)),
            pl.BlockSpec((bm, xk), lambda i: (i, 0)),
            full(in_dim, h1), full(xk, h1), full(1, h1),
            full(h1, h2), full(1, h2),
            full(h2, h3), full(1, h3),
            full(h3, od), full(1, od),
        ],
        out_specs=pl.BlockSpec((bm, od), lambda i: (i, 0)),
        out_shape=jax.ShapeDtypeStruct((_B, od), jnp.float32),
    )(emb, xc_pad, w1a, w1b, b1, w2, b2, w3, b3, w4, b4)


def kernel(x_categ, x_cont, embed_tables, W1, b1, W2, b2, W3, b3, W4, b4):
    tab_flat = embed_tables.reshape(_F * _V, _D)
    idx_flat = x_categ.reshape(_ROWS)
    emb = _sc_gather(tab_flat, idx_flat).reshape(_B, _F * _D)

    xc_pad = jnp.pad(x_cont, ((0, 0), (0, 3)))
    w1a = W1[:_F * _D]
    w1b = jnp.pad(W1[_F * _D:], ((0, 3), (0, 0)))
    logits = _tc_mlp(
        emb, xc_pad, w1a, w1b, b1.reshape(1, -1), W2, b2.reshape(1, -1),
        W3, b3.reshape(1, -1), W4, b4.reshape(1, -1))
    return logits


# speedup vs baseline: 7.8665x; 7.8665x over previous
import functools

import jax
import jax.numpy as jnp
from jax import lax
from jax.experimental import pallas as pl
from jax.experimental.pallas import tpu as pltpu
from jax.experimental.pallas import tpu_sc as plsc

_B = 16384
_F = 26
_V = 100000
_D = 16
_NC = 2
_NS = 16
_NW = _NC * _NS
_L = 16

_ROWS = _B * _F
_RPW = _ROWS // _NW
_CHUNK = 6656
_NCHUNK = _RPW // _CHUNK


def _sc_gather(tab_flat, idx_flat):
    mesh = plsc.VectorSubcoreMesh(
        core_axis_name="c", subcore_axis_name="s",
        num_cores=_NC, num_subcores=_NS)

    @functools.partial(
        pl.kernel,
        out_type=jax.ShapeDtypeStruct((_ROWS, _D), jnp.float32),
        mesh=mesh,
        scratch_types=[
            pltpu.VMEM((_CHUNK,), jnp.int32),
            pltpu.VMEM((_CHUNK, _D), jnp.float32),
            pltpu.SemaphoreType.DMA,
        ],
        compiler_params=pltpu.CompilerParams(use_tc_tiling_on_sc=False),
    )
    def body(tab_hbm, idx_hbm, out_hbm, idx_v, rows_v, sem):
        wid = lax.axis_index("s") * _NC + lax.axis_index("c")
        lane = lax.broadcasted_iota(jnp.int32, (_L,), 0)

        def do_chunk(c, _):
            gbase = wid * _RPW + c * _CHUNK
            pltpu.sync_copy(idx_hbm.at[pl.ds(gbase, _CHUNK)], idx_v)

            def add_off(i, _):
                p = i * _L + lane
                off = lax.rem(p, _F) * _V
                idx_v[pl.ds(i * _L, _L)] = idx_v[pl.ds(i * _L, _L)] + off
                return 0
            lax.fori_loop(0, _CHUNK // _L, add_off, 0, unroll=4)

            pltpu.async_copy(tab_hbm.at[idx_v], rows_v, sem).wait()
            pltpu.sync_copy(rows_v, out_hbm.at[pl.ds(gbase, _CHUNK)])
            return 0

        lax.fori_loop(0, _NCHUNK, do_chunk, 0)

    return body(tab_flat, idx_flat)


def _mlp_body(emb, xc, w1a, w1b, b1, w2, b2, w3, b3, w4, b4, out):
    h = jnp.dot(emb[...], w1a[...], preferred_element_type=jnp.float32)
    h += jnp.dot(xc[...], w1b[...], preferred_element_type=jnp.float32)
    h = jnp.maximum(h + b1[...], 0.0)
    h = jnp.maximum(
        jnp.dot(h, w2[...], preferred_element_type=jnp.float32) + b2[...], 0.0)
    h = jnp.maximum(
        jnp.dot(h, w3[...], preferred_element_type=jnp.float32) + b3[...], 0.0)
    out[...] = jnp.dot(h, w4[...], preferred_element_type=jnp.float32) + b4[...]


def _tc_mlp(emb, xc_pad, w1a, w1b, b1, w2, b2, w3, b3, w4, b4):
    bm = 1024
    in_dim = emb.shape[1]
    h1, h2, h3, od = w1a.shape[1], w2.shape[1], w3.shape[1], w4.shape[1]
    xk = xc_pad.shape[1]
    grid = (_B // bm,)
    full = lambda r, c: pl.BlockSpec((r, c), lambda i: (0, 0))
    return pl.pallas_call(
        _mlp_body,
        grid=grid,
        in_specs=[
            pl.BlockSpec((bm, in_dim), lambda i: (i, 0)),
            pl.BlockSpec((bm, xk), lambda i: (i, 0)),
            full(in_dim, h1), full(xk, h1), full(1, h1),
            full(h1, h2), full(1, h2),
            full(h2, h3), full(1, h3),
            full(h3, od), full(1, od),
        ],
        out_specs=pl.BlockSpec((bm, od), lambda i: (i, 0)),
        out_shape=jax.ShapeDtypeStruct((_B, od), jnp.float32),
    )(emb, xc_pad, w1a, w1b, b1, w2, b2, w3, b3, w4, b4)


def kernel(x_categ, x_cont, embed_tables, W1, b1, W2, b2, W3, b3, W4, b4):
    tab_flat = embed_tables.reshape(_F * _V, _D)
    idx_flat = x_categ.reshape(_ROWS)
    emb = _sc_gather(tab_flat, idx_flat).reshape(_B, _F * _D)

    xc_pad = jnp.pad(x_cont, ((0, 0), (0, 3)))
    w1a = W1[:_F * _D]
    w1b = jnp.pad(W1[_F * _D:], ((0, 3), (0, 0)))
    logits = _tc_mlp(
        emb, xc_pad, w1a, w1b, b1.reshape(1, -1), W2, b2.reshape(1, -1),
        W3, b3.reshape(1, -1), W4, b4.reshape(1, -1))
    return logits


# speedup vs baseline: 11.2441x; 1.4294x over previous
import functools

import jax
import jax.numpy as jnp
from jax import lax
from jax.experimental import pallas as pl
from jax.experimental.pallas import tpu as pltpu
from jax.experimental.pallas import tpu_sc as plsc

_B = 16384
_F = 26
_FG = 4
_FPG = 8
_V = 100000
_D = 16
_NC = 2
_NS = 16
_NW = _NC * _NS
_L = 16

_ROWS = _B * _FG * _FPG
_GROUP = _B * _FPG
_CHUNK = _GROUP // _NW
_VB = 2048


def _tr_body(x_ref, o_ref):
    o_ref[0] = x_ref[...].T


def _tc_transpose(tab2):
    nv = (_V + _VB - 1) // _VB
    return pl.pallas_call(
        _tr_body,
        grid=(_FG, nv),
        in_specs=[pl.BlockSpec((128, _VB), lambda r, v: (r, v))],
        out_specs=pl.BlockSpec((1, _VB, 128), lambda r, v: (r, v, 0)),
        out_shape=jax.ShapeDtypeStruct((_FG, _V, _FPG * _D), jnp.float32),
    )(tab2)


def _sc_gather(tab_lin, idx_flat):
    mesh = plsc.VectorSubcoreMesh(
        core_axis_name="c", subcore_axis_name="s",
        num_cores=_NC, num_subcores=_NS)

    @functools.partial(
        pl.kernel,
        out_type=jax.ShapeDtypeStruct((_ROWS, _D), jnp.float32),
        mesh=mesh,
        scratch_types=[
            pltpu.VMEM((_CHUNK,), jnp.int32),
            pltpu.VMEM((_CHUNK, _D), jnp.float32),
            pltpu.SemaphoreType.DMA,
        ],
        compiler_params=pltpu.CompilerParams(use_tc_tiling_on_sc=False),
    )
    def body(tab_hbm, idx_hbm, out_hbm, idx_v, rows_v, sem):
        wid = lax.axis_index("s") * _NC + lax.axis_index("c")
        lane = lax.broadcasted_iota(jnp.int32, (_L,), 0)

        for a in range(_FG):
            gbase = a * _GROUP + wid * _CHUNK
            pltpu.sync_copy(idx_hbm.at[pl.ds(gbase, _CHUNK)], idx_v)

            def add_off(i, _, a=a):
                fl = (i * _L + lane) & (_FPG - 1)
                row = idx_v[pl.ds(i * _L, _L)] * _FPG + fl + a * _FPG * _V
                if a == _FG - 1:
                    row = jnp.where(fl < _F - _FPG * a, row, 0)
                idx_v[pl.ds(i * _L, _L)] = row
                return 0
            lax.fori_loop(0, _CHUNK // _L, add_off, 0, unroll=4)

            pltpu.async_copy(tab_hbm.at[idx_v], rows_v, sem).wait()
            pltpu.sync_copy(rows_v, out_hbm.at[pl.ds(gbase, _CHUNK)])

    return body(tab_lin, idx_flat)


def _mlp_body(e4, xc, w0, w1, w2, w3, wb, b1, W2, b2, W3, b3,
              W4, b4, out):
    h = jnp.dot(e4[0], w0[...], preferred_element_type=jnp.float32)
    h += jnp.dot(e4[1], w1[...], preferred_element_type=jnp.float32)
    h += jnp.dot(e4[2], w2[...], preferred_element_type=jnp.float32)
    h += jnp.dot(e4[3], w3[...], preferred_element_type=jnp.float32)
    h += jnp.dot(xc[...], wb[...], preferred_element_type=jnp.float32)
    h = jnp.maximum(h + b1[...], 0.0)
    h = jnp.maximum(
        jnp.dot(h, W2[...], preferred_element_type=jnp.float32) + b2[...], 0.0)
    h = jnp.maximum(
        jnp.dot(h, W3[...], preferred_element_type=jnp.float32) + b3[...], 0.0)
    out[...] = jnp.dot(h, W4[...], preferred_element_type=jnp.float32) + b4[...]


def _tc_mlp(emb4, xc_pad, w1parts, w1b, b1, W2, b2, W3, b3, W4, b4):
    bm = 1024
    h1, h2, h3, od = W2.shape[0], W2.shape[1], W3.shape[1], W4.shape[1]
    xk = xc_pad.shape[1]
    grid = (_B // bm,)
    full = lambda r, c: pl.BlockSpec((r, c), lambda i: (0, 0))
    eb = pl.BlockSpec((_FG, bm, 128), lambda i: (0, i, 0))
    return pl.pallas_call(
        _mlp_body,
        grid=grid,
        in_specs=[eb, pl.BlockSpec((bm, xk), lambda i: (i, 0))]
        + [full(128, h1)] * _FG
        + [full(xk, h1), full(1, h1), full(h1, h2), full(1, h2),
           full(h2, h3), full(1, h3), full(h3, od), full(1, od)],
        out_specs=pl.BlockSpec((bm, od), lambda i: (i, 0)),
        out_shape=jax.ShapeDtypeStruct((_B, od), jnp.float32),
    )(emb4, xc_pad, *w1parts, w1b, b1, W2, b2, W3, b3, W4, b4)


def kernel(x_categ, x_cont, embed_tables, W1, b1, W2, b2, W3, b3, W4, b4):
    tab2 = jnp.transpose(embed_tables, (0, 2, 1)).reshape(_F * _D, _V)
    tab_big = _tc_transpose(tab2)
    tab_lin = tab_big.reshape(_FG * _V * _FPG, _D)

    x_pad = jnp.pad(x_categ, ((0, 0), (0, _FG * _FPG - _F)))
    idx_flat = x_pad.reshape(_B, _FG, _FPG).transpose(1, 0, 2).reshape(_ROWS)

    emb4 = _sc_gather(tab_lin, idx_flat).reshape(_FG, _B, _FPG * _D)

    xc_pad = jnp.pad(x_cont, ((0, 0), (0, 3)))
    w1parts = [W1[a * 128:(a + 1) * 128] for a in range(_FG - 1)]
    w1parts.append(jnp.concatenate(
        [W1[384:_F * _D], jnp.zeros((128 - 32, W1.shape[1]), W1.dtype)]))
    w1b = jnp.pad(W1[_F * _D:], ((0, 3), (0, 0)))
    logits = _tc_mlp(
        emb4, xc_pad, w1parts, w1b, b1.reshape(1, -1), W2, b2.reshape(1, -1),
        W3, b3.reshape(1, -1), W4, b4.reshape(1, -1))
    return logits
